# baseline (speedup 1.0000x reference)
import functools

import jax
import jax.numpy as jnp
from jax import lax
from jax.experimental import pallas as pl
from jax.experimental.pallas import tpu as pltpu
from jax.experimental.pallas import tpu_sc as plsc

_B = 1024
_P = 20
_V = 20
_D = 18432
_N = _B * _P
_NC = 2
_NS = 16
_NW = _NC * _NS
_BPW = _N // _NW
_CB = 2
_NBUF = 2
_NIT = _BPW // _CB
_NGRP = _NIT // _NBUF

_mesh = plsc.VectorSubcoreMesh(core_axis_name="c", subcore_axis_name="s")


@functools.partial(
    pl.kernel,
    out_type=jax.ShapeDtypeStruct((_N, _D), jnp.float32),
    mesh=_mesh,
    scratch_types=[
        pltpu.VMEM((_NIT, _CB), jnp.int32),
        pltpu.VMEM((_CB, _D), jnp.float32),
        pltpu.VMEM((_CB, _D), jnp.float32),
        pltpu.SemaphoreType.DMA,
        pltpu.SemaphoreType.DMA,
        pltpu.SemaphoreType.DMA,
        pltpu.SemaphoreType.DMA,
    ],
)
def _sc_lookup(idx_hbm, table_hbm, out_hbm,
               idx_v, buf0, buf1, gs0, gs1, ps0, ps1):
    wid = lax.axis_index("s") * _NC + lax.axis_index("c")
    base = wid * _BPW
    pltpu.sync_copy(idx_hbm.at[wid], idx_v)

    bufs = (buf0, buf1)
    gsem = (gs0, gs1)
    psem = (ps0, ps1)

    def gcopy(i, b):
        return pltpu.make_async_copy(
            table_hbm.at[idx_v.at[i]], bufs[b], gsem[b])

    def pcopy(i, b):
        return pltpu.make_async_copy(
            bufs[b], out_hbm.at[pl.ds(base + i * _CB, _CB)], psem[b])

    for b in range(_NBUF):
        gcopy(b, b).start()

    def body(grp, carry):
        for b in range(_NBUF):
            i = grp * _NBUF + b
            gcopy(i, b).wait()
            pcopy(i, b).start()
        for b in range(_NBUF):
            i = grp * _NBUF + b
            pcopy(i, b).wait()
            nxt = i + _NBUF

            @pl.when(nxt < _NIT)
            def _():
                gcopy(nxt, b).start()
        return carry

    lax.fori_loop(0, _NGRP, body, 0)


def kernel(prefix, embedding_table):
    idx3 = prefix.reshape(_NW, _NIT, _CB)
    out = _sc_lookup(idx3, embedding_table)
    return out.reshape(_B, _P, _D)


# speedup vs baseline: 1.2849x; 1.2849x over previous
import jax
import jax.numpy as jnp
from jax.experimental import pallas as pl

_B = 1024
_P = 20
_V = 20
_D = 18432
_BLK = 256
_CSPLIT = 2
_DC = _D // _CSPLIT


def _body(idx_ref, table_ref, out_ref):
    idx = idx_ref[0, 0]
    iota = jax.lax.broadcasted_iota(jnp.int32, (_BLK, _V), 1)
    onehot = (idx[:, None] == iota).astype(jnp.float32)
    out_ref[...] = jnp.dot(onehot, table_ref[...],
                           preferred_element_type=jnp.float32)


def kernel(prefix, embedding_table):
    n = _B * _P
    nblk = n // _BLK
    idx3 = prefix.reshape(nblk, 1, _BLK)
    out = pl.pallas_call(
        _body,
        grid=(nblk, _CSPLIT),
        in_specs=[
            pl.BlockSpec((1, 1, _BLK), lambda i, j: (i, 0, 0)),
            pl.BlockSpec((_V, _DC), lambda i, j: (0, j)),
        ],
        out_specs=pl.BlockSpec((_BLK, _DC), lambda i, j: (i, j)),
        out_shape=jax.ShapeDtypeStruct((n, _D), jnp.float32),
    )(idx3, embedding_table)
    return out.reshape(_B, _P, _D)


# speedup vs baseline: 1.2890x; 1.0032x over previous
import functools

import jax
import jax.numpy as jnp
from jax import lax
from jax.experimental import pallas as pl
from jax.experimental.pallas import tpu as pltpu
from jax.experimental.pallas import tpu_sc as plsc

_B = 1024
_P = 20
_V = 20
_D = 18432
_N = _B * _P
_NCOL = 4
_NROW = 8
_DS = _D // _NCOL
_RPW = _N // _NROW
_SB = 640
_NSB = _RPW // _SB

_mesh = plsc.VectorSubcoreMesh(core_axis_name="c", subcore_axis_name="s")


@functools.partial(
    pl.kernel,
    out_type=jax.ShapeDtypeStruct((_N, _D), jnp.float32),
    mesh=_mesh,
    scratch_types=[
        pltpu.VMEM((_SB,), jnp.int32),
        pltpu.VMEM((_V, _DS), jnp.float32),
        pltpu.SemaphoreType.DMA,
    ],
)
def _sc_lookup(idx_hbm, table_hbm, out_hbm, idx_v, tslice, psem):
    cid = lax.axis_index("c")
    sid = lax.axis_index("s")
    colg = sid // _NCOL
    rowg = (sid % _NCOL) + _NCOL * cid
    col0 = colg * _DS
    row0 = rowg * _RPW

    pltpu.sync_copy(table_hbm.at[:, pl.ds(col0, _DS)], tslice)

    def sb_body(sb, carry):
        pltpu.sync_copy(idx_hbm.at[rowg, sb], idx_v)

        def grp_body(g, carry2):
            vec = idx_v[pl.ds(g * 16, 16)]
            r0 = row0 + sb * _SB + g * 16
            for l in range(16):
                pltpu.make_async_copy(
                    tslice.at[pl.ds(vec[l], 1)],
                    out_hbm.at[pl.ds(r0 + l, 1), pl.ds(col0, _DS)],
                    psem).start()
            return carry2

        lax.fori_loop(0, _SB // 16, grp_body, 0)
        return carry

    lax.fori_loop(0, _NSB, sb_body, 0)

    def drain(j, carry):
        pltpu.make_async_copy(
            tslice.at[pl.ds(0, 1)],
            out_hbm.at[pl.ds(row0, 1), pl.ds(col0, _DS)],
            psem).wait()
        return carry

    lax.fori_loop(0, _RPW, drain, 0)


def kernel(prefix, embedding_table):
    idx3 = prefix.reshape(_NROW, _NSB, _SB)
    out = _sc_lookup(idx3, embedding_table)
    return out.reshape(_B, _P, _D)


# speedup vs baseline: 1.2899x; 1.0007x over previous
import functools

import jax
import jax.numpy as jnp
from jax import lax
from jax.experimental import pallas as pl
from jax.experimental.pallas import tpu as pltpu
from jax.experimental.pallas import tpu_sc as plsc

_B = 1024
_P = 20
_V = 20
_D = 18432
_N = _B * _P
_NCOL = 4
_NROW = 8
_DS = _D // _NCOL
_RPW = _N // _NROW
_NGRP = _RPW // 16

_mesh = plsc.VectorSubcoreMesh(core_axis_name="c", subcore_axis_name="s")


@functools.partial(
    pl.kernel,
    out_type=jax.ShapeDtypeStruct((_N, _D), jnp.float32),
    mesh=_mesh,
    scratch_types=[
        pltpu.VMEM((_RPW,), jnp.int32),
        pltpu.VMEM((_V, _DS), jnp.float32),
        pltpu.SemaphoreType.DMA,
    ],
)
def _sc_lookup(idx_hbm, table_hbm, out_hbm, idx_v, tslice, psem):
    cid = lax.axis_index("c")
    sid = lax.axis_index("s")
    colg = sid // _NCOL
    rowg = (sid % _NCOL) + _NCOL * cid
    col0 = colg * _DS
    row0 = rowg * _RPW

    pltpu.sync_copy(table_hbm.at[:, pl.ds(col0, _DS)], tslice)
    pltpu.sync_copy(idx_hbm.at[rowg], idx_v)

    def grp_body(g, carry):
        vec = idx_v[pl.ds(g * 16, 16)]
        r0 = row0 + g * 16
        for l in range(16):
            pltpu.make_async_copy(
                tslice.at[pl.ds(vec[l], 1)],
                out_hbm.at[pl.ds(r0 + l, 1), pl.ds(col0, _DS)],
                psem).start()
        return carry

    lax.fori_loop(0, _NGRP, grp_body, 0)

    def drain(j, carry):
        pltpu.make_async_copy(
            tslice.at[pl.ds(0, 1)],
            out_hbm.at[pl.ds(row0, 1), pl.ds(col0, _DS)],
            psem).wait()
        return carry

    lax.fori_loop(0, _RPW, drain, 0)


def kernel(prefix, embedding_table):
    idx2 = prefix.reshape(_NROW, _RPW)
    out = _sc_lookup(idx2, embedding_table)
    return out.reshape(_B, _P, _D)
